# Initial kernel scaffold; baseline (speedup 1.0000x reference)
import jax
import jax.numpy as jnp
from jax.experimental import pallas as pl


def kernel(Ll0, Lu0, D0, adD0, x0, theta0_1, bias0_1, theta0_2, bias0_2, theta0_3, bias0_3, Ll1, Lu1, D1, adD1, x1, theta1_1, bias1_1, theta1_2, bias1_2, theta1_3, bias1_3, Ll2, Lu2, D2, adD2, x2, theta2_1, bias2_1, theta2_2, bias2_2, theta2_3, bias2_3, Ll3, Lu3, D3, adD3, x3, theta3_1, bias3_1, theta3_2, bias3_2, theta3_3, bias3_3, Ll4, Lu4, D4, adD4, x4, theta4_1, bias4_1, theta4_2, bias4_2, theta4_3, bias4_3, Ll5, Lu5, D5, adD5, x5, theta5_1, bias5_1, theta5_2, bias5_2, theta5_3, bias5_3):
    raise NotImplementedError("write your pallas kernel here")


# speedup vs baseline: 1.2947x; 1.2947x over previous
import functools

import jax
import jax.numpy as jnp
from jax.experimental import pallas as pl

_M = 2048
_SLOPE = 0.01


def _lrelu(y):
    return jnp.where(y >= 0, y, _SLOPE * y)


def _scnn_dim_kernel(ll_ref, lu_ref, x_ref, w1_ref, w2_ref, w3_ref,
                     b1_ref, b2_ref, b3_ref, out_ref):
    ll = ll_ref[...]
    lu = lu_ref[...]

    def lap(mat, v):
        return jnp.dot(mat, v, preferred_element_type=jnp.float32)

    def mix(terms, w_ref, b_ref):
        acc = jnp.dot(terms[0], w_ref[0], preferred_element_type=jnp.float32)
        for k in range(1, len(terms)):
            acc = acc + jnp.dot(terms[k], w_ref[k],
                                preferred_element_type=jnp.float32)
        return acc + b_ref[...]

    xc = x_ref[...]
    t1 = lap(ll, xc)
    t2 = lap(lu, xc)
    t3 = lap(lu, t2)
    u = _lrelu(mix([xc, t1, t2, t3], w1_ref, b1_ref))

    u1 = lap(ll, u)
    u2 = lap(lu, u)
    u3 = lap(lu, u2)
    v = _lrelu(mix([u, u1, u2, u3], w2_ref, b2_ref))

    v1 = lap(ll, v)
    v2 = lap(lu, v)
    v3 = lap(lu, v2)
    out_ref[...] = mix([v, v1, v2, v3], w3_ref, b3_ref)


@functools.partial(jax.jit, static_argnames=())
def _run_dim(ll, lu, x, th1, b1, th2, b2, th3, b3):
    xc = x.reshape(_M, 1)
    w1 = jnp.transpose(th1, (2, 1, 0))
    w2 = jnp.transpose(th2, (2, 1, 0))
    w3 = jnp.transpose(th3, (2, 1, 0))
    b1r = b1.reshape(1, -1)
    b2r = b2.reshape(1, -1)
    b3r = b3.reshape(1, -1)
    out = pl.pallas_call(
        _scnn_dim_kernel,
        out_shape=jax.ShapeDtypeStruct((_M, 1), jnp.float32),
    )(ll, lu, xc, w1, w2, w3, b1r, b2r, b3r)
    return out.reshape(1, 1, _M)


def kernel(Ll0, Lu0, D0, adD0, x0, theta0_1, bias0_1, theta0_2, bias0_2, theta0_3, bias0_3,
           Ll1, Lu1, D1, adD1, x1, theta1_1, bias1_1, theta1_2, bias1_2, theta1_3, bias1_3,
           Ll2, Lu2, D2, adD2, x2, theta2_1, bias2_1, theta2_2, bias2_2, theta2_3, bias2_3,
           Ll3, Lu3, D3, adD3, x3, theta3_1, bias3_1, theta3_2, bias3_2, theta3_3, bias3_3,
           Ll4, Lu4, D4, adD4, x4, theta4_1, bias4_1, theta4_2, bias4_2, theta4_3, bias4_3,
           Ll5, Lu5, D5, adD5, x5, theta5_1, bias5_1, theta5_2, bias5_2, theta5_3, bias5_3):
    inp = locals()
    outs = []
    for d in range(6):
        outs.append(_run_dim(
            inp['Ll%d' % d], inp['Lu%d' % d], inp['x%d' % d],
            inp['theta%d_1' % d], inp['bias%d_1' % d],
            inp['theta%d_2' % d], inp['bias%d_2' % d],
            inp['theta%d_3' % d], inp['bias%d_3' % d]))
    return tuple(outs)
